# Initial kernel scaffold; baseline (speedup 1.0000x reference)
import jax
import jax.numpy as jnp
from jax.experimental import pallas as pl


def kernel(h_router, h_packet, W_p, b_p, W_c, b_c, edge_output, edge_input, edge_pass):
    raise NotImplementedError("write your pallas kernel here")


# speedup vs baseline: 14.5843x; 14.5843x over previous
import functools
import jax
import jax.numpy as jnp
from jax import lax
from jax.experimental import pallas as pl
from jax.experimental.pallas import tpu as pltpu
from jax.experimental.pallas import tpu_sc as plsc

H = 64
HH = 32
NCH = 10000
NRT = 10000
NPK = 10000
ACC_ROWS = 10112
ZROWS = 632
OROWS = 624

CHUNK = 128
NCH_H = 49
NCH_P = 5
EH_PAD = 16 * NCH_H * CHUNK
EP_PAD = 32 * NCH_P * CHUNK

_mesh = plsc.VectorSubcoreMesh(core_axis_name="c", subcore_axis_name="s")
_sc_params = pltpu.CompilerParams(use_tc_tiling_on_sc=False)


@functools.partial(
    pl.kernel,
    out_type=[
        jax.ShapeDtypeStruct((2, NCH, H), jnp.float32),
        jax.ShapeDtypeStruct((2, NCH, H + 16), jnp.float32),
    ],
    mesh=_mesh,
    compiler_params=_sc_params,
    scratch_types=[
        pltpu.VMEM_SHARED((ACC_ROWS, H), jnp.float32),
        pltpu.VMEM_SHARED((ACC_ROWS, H + 16), jnp.float32),
        pltpu.VMEM((NCH_H, CHUNK), jnp.int32),
        pltpu.VMEM((NCH_H, CHUNK), jnp.int32),
        pltpu.VMEM((NCH_P, CHUNK), jnp.int32),
        pltpu.VMEM((NCH_P, CHUNK), jnp.int32),
        pltpu.VMEM((CHUNK, H), jnp.float32),
        pltpu.VMEM((CHUNK, H + 16), jnp.float32),
    ],
)
def _sc_bin_channels(hr, hpx, ghs, ghd, gps, gpd, z64, z80,
                     hio, sp,
                     acc64, acc80, isrc, idst, psrc, pdst, rows, prow):
    c = lax.axis_index("c")
    s = lax.axis_index("s")
    z0 = s * ZROWS
    pltpu.sync_copy(z64.at[pl.ds(z0, ZROWS)], acc64.at[pl.ds(z0, ZROWS)])
    pltpu.sync_copy(z80.at[pl.ds(z0, ZROWS)], acc80.at[pl.ds(z0, ZROWS)])
    pltpu.sync_copy(ghs.at[c, s], isrc)
    pltpu.sync_copy(ghd.at[c, s], idst)
    pltpu.sync_copy(gps.at[c, s], psrc)
    pltpu.sync_copy(gpd.at[c, s], pdst)
    plsc.subcore_barrier()

    def hstep(k, carry):
        pltpu.sync_copy(hr.at[isrc.at[k]], rows)
        pltpu.sync_copy(rows, acc64.at[idst.at[k]], add=True)
        return carry

    lax.fori_loop(0, NCH_H, hstep, 0, unroll=False)

    def pstep(k, carry):
        pltpu.sync_copy(hpx.at[psrc.at[k]], prow)
        pltpu.sync_copy(prow, acc80.at[pdst.at[k]], add=True)
        return carry

    lax.fori_loop(0, NCH_P, pstep, 0, unroll=False)
    plsc.subcore_barrier()

    r0 = s * OROWS

    @pl.when(s < 15)
    def _():
        pltpu.sync_copy(acc64.at[pl.ds(r0, OROWS)], hio.at[c, pl.ds(r0, OROWS)])
        pltpu.sync_copy(acc80.at[pl.ds(r0, OROWS)], sp.at[c, pl.ds(r0, OROWS)])

    @pl.when(s == 15)
    def _():
        pltpu.sync_copy(acc64.at[pl.ds(15 * OROWS, 640)],
                        hio.at[c, pl.ds(15 * OROWS, 640)])
        pltpu.sync_copy(acc80.at[pl.ds(15 * OROWS, 640)],
                        sp.at[c, pl.ds(15 * OROWS, 640)])


@functools.partial(
    pl.kernel,
    out_type=jax.ShapeDtypeStruct((2, NRT, HH), jnp.float32),
    mesh=_mesh,
    compiler_params=_sc_params,
    scratch_types=[
        pltpu.VMEM_SHARED((ACC_ROWS, HH), jnp.float32),
        pltpu.VMEM((NCH_H, CHUNK), jnp.int32),
        pltpu.VMEM((NCH_H, CHUNK), jnp.int32),
        pltpu.VMEM((CHUNK, HH), jnp.float32),
    ],
)
def _sc_bin_routers(mcat, gbs, gbd, z32, mr, acc32, isrc, idst, rows):
    c = lax.axis_index("c")
    s = lax.axis_index("s")
    z0 = s * ZROWS
    pltpu.sync_copy(z32.at[pl.ds(z0, ZROWS)], acc32.at[pl.ds(z0, ZROWS)])
    pltpu.sync_copy(gbs.at[c, s], isrc)
    pltpu.sync_copy(gbd.at[c, s], idst)
    plsc.subcore_barrier()

    def step(k, carry):
        pltpu.sync_copy(mcat.at[isrc.at[k]], rows)
        pltpu.sync_copy(rows, acc32.at[idst.at[k]], add=True)
        return carry

    lax.fori_loop(0, NCH_H, step, 0, unroll=False)
    plsc.subcore_barrier()

    r0 = s * OROWS

    @pl.when(s < 15)
    def _():
        pltpu.sync_copy(acc32.at[pl.ds(r0, OROWS)], mr.at[c, pl.ds(r0, OROWS)])

    @pl.when(s == 15)
    def _():
        pltpu.sync_copy(acc32.at[pl.ds(15 * OROWS, 640)],
                        mr.at[c, pl.ds(15 * OROWS, 640)])


TCT = 400


def _tc_dense_body(hio, sp, wp, bp, mio):
    sext = sp[0] + sp[1]
    sv = sext[:, :H]
    cnt = sext[:, H:H + 1]
    e = jnp.dot(sv, wp[...], preferred_element_type=jnp.float32) + cnt * bp[...]
    e3 = e.reshape(TCT, H, HH)
    hin = hio[0]
    hout = hio[1]
    mio[0] = jnp.sum(e3 * hin[:, :, None], axis=1)
    mio[1] = jnp.sum(e3 * hout[:, :, None], axis=1)


def _tc_dense(hio, sp, wp, bp2):
    return pl.pallas_call(
        _tc_dense_body,
        grid=(NCH // TCT,),
        in_specs=[
            pl.BlockSpec((2, TCT, H), lambda t: (0, t, 0)),
            pl.BlockSpec((2, TCT, H + 16), lambda t: (0, t, 0)),
            pl.BlockSpec((H, H * HH), lambda t: (0, 0)),
            pl.BlockSpec((1, H * HH), lambda t: (0, 0)),
        ],
        out_specs=pl.BlockSpec((2, TCT, HH), lambda t: (0, t, 0)),
        out_shape=jax.ShapeDtypeStruct((2, NCH, HH), jnp.float32),
    )(hio, sp, wp, bp2)


TCR = 2000


def _tc_relu_body(mr, om):
    om[...] = jnp.maximum(
        jnp.concatenate([mr[0], mr[1]], axis=1), 0.0)


def _tc_relu(mr):
    return pl.pallas_call(
        _tc_relu_body,
        grid=(NRT // TCR,),
        in_specs=[pl.BlockSpec((2, TCR, HH), lambda t: (0, t, 0))],
        out_specs=pl.BlockSpec((TCR, H), lambda t: (t, 0)),
        out_shape=jax.ShapeDtypeStruct((NRT, H), jnp.float32),
    )(mr)


def _pad_idx(x, pad_len, pad_val):
    return jnp.concatenate(
        [x, jnp.full((pad_len,), pad_val, jnp.int32)])


def kernel(h_router, h_packet, W_p, b_p, W_c, b_c, edge_output, edge_input, edge_pass):
    del W_c, b_c

    hpx = jnp.concatenate(
        [h_packet,
         jnp.ones((NPK, 1), jnp.float32),
         jnp.zeros((NPK, 15), jnp.float32)], axis=1)

    padh = EH_PAD - edge_output.shape[1]
    ghs = jnp.stack([
        _pad_idx(edge_output[0], padh, 0),
        _pad_idx(edge_input[1], padh, 0),
    ]).reshape(2, 16, NCH_H, CHUNK)
    ghd = jnp.stack([
        _pad_idx(edge_output[1], padh, NCH),
        _pad_idx(edge_input[0], padh, NCH),
    ]).reshape(2, 16, NCH_H, CHUNK)

    padp = EP_PAD - edge_pass.shape[1]
    gps = _pad_idx(edge_pass[0], padp, 0).reshape(2, 16, NCH_P, CHUNK)
    gpd = _pad_idx(edge_pass[1], padp, NCH).reshape(2, 16, NCH_P, CHUNK)

    z64 = jnp.zeros((ACC_ROWS, H), jnp.float32)
    z80 = jnp.zeros((ACC_ROWS, H + 16), jnp.float32)
    z32 = jnp.zeros((ACC_ROWS, HH), jnp.float32)

    hio, sp = _sc_bin_channels(h_router, hpx, ghs, ghd, gps, gpd, z64, z80)

    mio = _tc_dense(hio, sp, W_p, b_p.reshape(1, H * HH))
    mcat = mio.reshape(2 * NCH, HH)

    gbs = jnp.stack([
        _pad_idx(edge_input[0], padh, 0),
        _pad_idx(edge_output[1], padh, 0) + NCH,
    ]).reshape(2, 16, NCH_H, CHUNK)
    gbd = jnp.stack([
        _pad_idx(edge_input[1], padh, NRT),
        _pad_idx(edge_output[0], padh, NRT),
    ]).reshape(2, 16, NCH_H, CHUNK)

    mr = _sc_bin_routers(mcat, gbs, gbd, z32)
    return _tc_relu(mr)


# speedup vs baseline: 27.2407x; 1.8678x over previous
import functools
import jax
import jax.numpy as jnp
from jax import lax
from jax.experimental import pallas as pl
from jax.experimental.pallas import tpu as pltpu
from jax.experimental.pallas import tpu_sc as plsc

H = 64
HH = 32
NCH = 10000
NRT = 10000
NPK = 10000
ACC_ROWS = 10112
ZROWS = 632
OROWS = 624

CHUNK = 128
NCH_H = 50
CHUNK_P = 64
NCH_P = 10
EH_PAD = 16 * NCH_H * CHUNK
EP_PAD = 32 * NCH_P * CHUNK_P

_mesh = plsc.VectorSubcoreMesh(core_axis_name="c", subcore_axis_name="s")
_sc_params = pltpu.CompilerParams(use_tc_tiling_on_sc=False)


@functools.partial(
    pl.kernel,
    out_type=[
        jax.ShapeDtypeStruct((2, NCH, H), jnp.float32),
        jax.ShapeDtypeStruct((2, NCH, H + 16), jnp.float32),
    ],
    mesh=_mesh,
    compiler_params=_sc_params,
    scratch_types=[
        pltpu.VMEM_SHARED((ACC_ROWS, H), jnp.float32),
        pltpu.VMEM_SHARED((ACC_ROWS, H + 16), jnp.float32),
        pltpu.VMEM((NCH_H, CHUNK), jnp.int32),
        pltpu.VMEM((NCH_H, CHUNK), jnp.int32),
        pltpu.VMEM((NCH_P, CHUNK_P), jnp.int32),
        pltpu.VMEM((NCH_P, CHUNK_P), jnp.int32),
        pltpu.VMEM((CHUNK, H), jnp.float32),
        pltpu.VMEM((CHUNK, H), jnp.float32),
        pltpu.VMEM((CHUNK_P, H + 16), jnp.float32),
        pltpu.SemaphoreType.DMA,
        pltpu.SemaphoreType.DMA,
    ],
)
def _sc_bin_channels(hr, hpx, ghs, ghd, gps, gpd, z64, z80,
                     hio, sp,
                     acc64, acc80, isrc, idst, psrc, pdst,
                     rows_a, rows_b, prow, gsem_a, gsem_b):
    c = lax.axis_index("c")
    s = lax.axis_index("s")
    z0 = s * ZROWS
    pltpu.sync_copy(z64.at[pl.ds(z0, ZROWS)], acc64.at[pl.ds(z0, ZROWS)])
    pltpu.sync_copy(z80.at[pl.ds(z0, ZROWS)], acc80.at[pl.ds(z0, ZROWS)])
    pltpu.sync_copy(ghs.at[c, s], isrc)
    pltpu.sync_copy(ghd.at[c, s], idst)
    pltpu.sync_copy(gps.at[c, s], psrc)
    pltpu.sync_copy(gpd.at[c, s], pdst)
    plsc.subcore_barrier()

    def _gwait(buf, sem):
        pltpu.make_async_copy(hr.at[isrc.at[0]], buf, sem).wait()

    pltpu.async_copy(hr.at[isrc.at[0]], rows_a, gsem_a)

    def hstep(i, carry):
        k0 = 2 * i
        k1 = k0 + 1
        _gwait(rows_a, gsem_a)
        pltpu.async_copy(hr.at[isrc.at[k1]], rows_b, gsem_b)
        pltpu.sync_copy(rows_a, acc64.at[idst.at[k0]], add=True)
        _gwait(rows_b, gsem_b)
        k2 = jnp.minimum(k0 + 2, NCH_H - 1)
        pltpu.async_copy(hr.at[isrc.at[k2]], rows_a, gsem_a)
        pltpu.sync_copy(rows_b, acc64.at[idst.at[k1]], add=True)
        return carry

    lax.fori_loop(0, NCH_H // 2, hstep, 0, unroll=False)
    _gwait(rows_a, gsem_a)

    def pstep(k, carry):
        pltpu.sync_copy(hpx.at[psrc.at[k]], prow)
        pltpu.sync_copy(prow, acc80.at[pdst.at[k]], add=True)
        return carry

    lax.fori_loop(0, NCH_P, pstep, 0, unroll=False)
    plsc.subcore_barrier()

    r0 = s * OROWS

    @pl.when(s < 15)
    def _():
        pltpu.sync_copy(acc64.at[pl.ds(r0, OROWS)], hio.at[c, pl.ds(r0, OROWS)])
        pltpu.sync_copy(acc80.at[pl.ds(r0, OROWS)], sp.at[c, pl.ds(r0, OROWS)])

    @pl.when(s == 15)
    def _():
        pltpu.sync_copy(acc64.at[pl.ds(15 * OROWS, 640)],
                        hio.at[c, pl.ds(15 * OROWS, 640)])
        pltpu.sync_copy(acc80.at[pl.ds(15 * OROWS, 640)],
                        sp.at[c, pl.ds(15 * OROWS, 640)])


@functools.partial(
    pl.kernel,
    out_type=jax.ShapeDtypeStruct((2, NRT, HH), jnp.float32),
    mesh=_mesh,
    compiler_params=_sc_params,
    scratch_types=[
        pltpu.VMEM_SHARED((ACC_ROWS, HH), jnp.float32),
        pltpu.VMEM((NCH_H, CHUNK), jnp.int32),
        pltpu.VMEM((NCH_H, CHUNK), jnp.int32),
        pltpu.VMEM((CHUNK, HH), jnp.float32),
        pltpu.VMEM((CHUNK, HH), jnp.float32),
        pltpu.SemaphoreType.DMA,
        pltpu.SemaphoreType.DMA,
    ],
)
def _sc_bin_routers(mcat, gbs, gbd, z32, mr, acc32, isrc, idst,
                    rows_a, rows_b, gsem_a, gsem_b):
    c = lax.axis_index("c")
    s = lax.axis_index("s")
    z0 = s * ZROWS
    pltpu.sync_copy(z32.at[pl.ds(z0, ZROWS)], acc32.at[pl.ds(z0, ZROWS)])
    pltpu.sync_copy(gbs.at[c, s], isrc)
    pltpu.sync_copy(gbd.at[c, s], idst)
    plsc.subcore_barrier()

    def _gwait(buf, sem):
        pltpu.make_async_copy(mcat.at[isrc.at[0]], buf, sem).wait()

    pltpu.async_copy(mcat.at[isrc.at[0]], rows_a, gsem_a)

    def step(i, carry):
        k0 = 2 * i
        k1 = k0 + 1
        _gwait(rows_a, gsem_a)
        pltpu.async_copy(mcat.at[isrc.at[k1]], rows_b, gsem_b)
        pltpu.sync_copy(rows_a, acc32.at[idst.at[k0]], add=True)
        _gwait(rows_b, gsem_b)
        k2 = jnp.minimum(k0 + 2, NCH_H - 1)
        pltpu.async_copy(mcat.at[isrc.at[k2]], rows_a, gsem_a)
        pltpu.sync_copy(rows_b, acc32.at[idst.at[k1]], add=True)
        return carry

    lax.fori_loop(0, NCH_H // 2, step, 0, unroll=False)
    _gwait(rows_a, gsem_a)
    plsc.subcore_barrier()

    r0 = s * OROWS

    @pl.when(s < 15)
    def _():
        pltpu.sync_copy(acc32.at[pl.ds(r0, OROWS)], mr.at[c, pl.ds(r0, OROWS)])

    @pl.when(s == 15)
    def _():
        pltpu.sync_copy(acc32.at[pl.ds(15 * OROWS, 640)],
                        mr.at[c, pl.ds(15 * OROWS, 640)])


TCT = 400

_RX_NP = None
_FX_NP = None


def _expand_fold_mats():
    global _RX_NP, _FX_NP
    if _RX_NP is None:
        import numpy as _np
        r = _np.zeros((H, H * HH), _np.float32)
        f = _np.zeros((H * HH, HH), _np.float32)
        for i in range(H):
            r[i, i * HH:(i + 1) * HH] = 1.0
            f[i * HH:(i + 1) * HH, :] = _np.eye(HH, dtype=_np.float32)
        _RX_NP, _FX_NP = r, f
    return _RX_NP, _FX_NP


def _tc_dense_body(hio, sp, wp, bp, rx, fx, mio):
    sext = sp[0] + sp[1]
    sv = sext[:, :H]
    cnt = sext[:, H:H + 1]
    e = jnp.dot(sv, wp[...], preferred_element_type=jnp.float32) + cnt * bp[...]
    hin_x = jnp.dot(hio[0], rx[...], preferred_element_type=jnp.float32)
    hout_x = jnp.dot(hio[1], rx[...], preferred_element_type=jnp.float32)
    mio[0] = jnp.dot(e * hin_x, fx[...], preferred_element_type=jnp.float32)
    mio[1] = jnp.dot(e * hout_x, fx[...], preferred_element_type=jnp.float32)


def _tc_dense(hio, sp, wp, bp2):
    rx, fx = _expand_fold_mats()
    return pl.pallas_call(
        _tc_dense_body,
        grid=(NCH // TCT,),
        in_specs=[
            pl.BlockSpec((2, TCT, H), lambda t: (0, t, 0)),
            pl.BlockSpec((2, TCT, H + 16), lambda t: (0, t, 0)),
            pl.BlockSpec((H, H * HH), lambda t: (0, 0)),
            pl.BlockSpec((1, H * HH), lambda t: (0, 0)),
            pl.BlockSpec((H, H * HH), lambda t: (0, 0)),
            pl.BlockSpec((H * HH, HH), lambda t: (0, 0)),
        ],
        out_specs=pl.BlockSpec((2, TCT, HH), lambda t: (0, t, 0)),
        out_shape=jax.ShapeDtypeStruct((2, NCH, HH), jnp.float32),
    )(hio, sp, wp, bp2, jnp.asarray(rx), jnp.asarray(fx))


TCR = 2000


def _tc_relu_body(mr, om):
    om[...] = jnp.maximum(
        jnp.concatenate([mr[0], mr[1]], axis=1), 0.0)


def _tc_relu(mr):
    return pl.pallas_call(
        _tc_relu_body,
        grid=(NRT // TCR,),
        in_specs=[pl.BlockSpec((2, TCR, HH), lambda t: (0, t, 0))],
        out_specs=pl.BlockSpec((TCR, H), lambda t: (t, 0)),
        out_shape=jax.ShapeDtypeStruct((NRT, H), jnp.float32),
    )(mr)


def _pad_idx(x, pad_len, pad_val):
    return jnp.concatenate(
        [x, jnp.full((pad_len,), pad_val, jnp.int32)])


def kernel(h_router, h_packet, W_p, b_p, W_c, b_c, edge_output, edge_input, edge_pass):
    del W_c, b_c

    hpx = jnp.concatenate(
        [h_packet,
         jnp.ones((NPK, 1), jnp.float32),
         jnp.zeros((NPK, 15), jnp.float32)], axis=1)

    padh = EH_PAD - edge_output.shape[1]
    ghs = jnp.stack([
        _pad_idx(edge_output[0], padh, 0),
        _pad_idx(edge_input[1], padh, 0),
    ]).reshape(2, 16, NCH_H, CHUNK)
    ghd = jnp.stack([
        _pad_idx(edge_output[1], padh, NCH),
        _pad_idx(edge_input[0], padh, NCH),
    ]).reshape(2, 16, NCH_H, CHUNK)

    padp = EP_PAD - edge_pass.shape[1]
    gps = _pad_idx(edge_pass[0], padp, 0).reshape(2, 16, NCH_P, CHUNK_P)
    gpd = _pad_idx(edge_pass[1], padp, NCH).reshape(2, 16, NCH_P, CHUNK_P)

    z64 = jnp.zeros((ACC_ROWS, H), jnp.float32)
    z80 = jnp.zeros((ACC_ROWS, H + 16), jnp.float32)
    z32 = jnp.zeros((ACC_ROWS, HH), jnp.float32)

    hio, sp = _sc_bin_channels(h_router, hpx, ghs, ghd, gps, gpd, z64, z80)

    mio = _tc_dense(hio, sp, W_p, b_p.reshape(1, H * HH))
    mcat = mio.reshape(2 * NCH, HH)

    gbs = jnp.stack([
        _pad_idx(edge_input[0], padh, 0),
        _pad_idx(edge_output[1], padh, 0) + NCH,
    ]).reshape(2, 16, NCH_H, CHUNK)
    gbd = jnp.stack([
        _pad_idx(edge_input[1], padh, NRT),
        _pad_idx(edge_output[0], padh, NRT),
    ]).reshape(2, 16, NCH_H, CHUNK)

    mr = _sc_bin_routers(mcat, gbs, gbd, z32)
    return _tc_relu(mr)


# speedup vs baseline: 29.5091x; 1.0833x over previous
import functools
import jax
import jax.numpy as jnp
from jax import lax
from jax.experimental import pallas as pl
from jax.experimental.pallas import tpu as pltpu
from jax.experimental.pallas import tpu_sc as plsc

H = 64
HH = 32
NCH = 10000
NRT = 10000
NPK = 10000
ACC_ROWS = 10000
OROWS = 624

CHUNK = 125
NCH_H = 50
CHUNK_P = 125
NCH_P = 5

_mesh = plsc.VectorSubcoreMesh(core_axis_name="c", subcore_axis_name="s")
_sc_params = pltpu.CompilerParams(use_tc_tiling_on_sc=False)


@functools.partial(
    pl.kernel,
    out_type=[
        jax.ShapeDtypeStruct((2, NCH, H), jnp.float32),
        jax.ShapeDtypeStruct((2, NCH, H + 16), jnp.float32),
    ],
    mesh=_mesh,
    compiler_params=_sc_params,
    scratch_types=[
        pltpu.VMEM_SHARED((ACC_ROWS, H), jnp.float32),
        pltpu.VMEM_SHARED((ACC_ROWS, H + 16), jnp.float32),
        pltpu.VMEM((NCH_H, CHUNK), jnp.int32),
        pltpu.VMEM((NCH_H, CHUNK), jnp.int32),
        pltpu.VMEM((NCH_P, CHUNK_P), jnp.int32),
        pltpu.VMEM((NCH_P, CHUNK_P), jnp.int32),
        pltpu.VMEM((CHUNK, H), jnp.float32),
        pltpu.VMEM((CHUNK, H), jnp.float32),
        pltpu.VMEM((CHUNK_P, H + 16), jnp.float32),
        pltpu.SemaphoreType.DMA,
        pltpu.SemaphoreType.DMA,
    ],
)
def _sc_bin_channels(hr, hpx, eo_r, ei_r, ep_r, z64, z80,
                     hio, sp,
                     acc64, acc80, isrc, idst, psrc, pdst,
                     rows_a, rows_b, prow, gsem_a, gsem_b):
    c = lax.axis_index("c")
    s = lax.axis_index("s")
    z0 = s * OROWS

    @pl.when(s < 15)
    def _():
        pltpu.sync_copy(z64.at[pl.ds(z0, OROWS)], acc64.at[pl.ds(z0, OROWS)])
        pltpu.sync_copy(z80.at[pl.ds(z0, OROWS)], acc80.at[pl.ds(z0, OROWS)])

    @pl.when(s == 15)
    def _():
        pltpu.sync_copy(z64.at[pl.ds(15 * OROWS, 640)],
                        acc64.at[pl.ds(15 * OROWS, 640)])
        pltpu.sync_copy(z80.at[pl.ds(15 * OROWS, 640)],
                        acc80.at[pl.ds(15 * OROWS, 640)])

    @pl.when(c == 0)
    def _():
        pltpu.sync_copy(eo_r.at[0, s], isrc)
        pltpu.sync_copy(eo_r.at[1, s], idst)

    @pl.when(c == 1)
    def _():
        pltpu.sync_copy(ei_r.at[1, s], isrc)
        pltpu.sync_copy(ei_r.at[0, s], idst)

    pltpu.sync_copy(ep_r.at[0, c, s], psrc)
    pltpu.sync_copy(ep_r.at[1, c, s], pdst)
    plsc.subcore_barrier()

    def _gwait(buf, sem):
        pltpu.make_async_copy(hr.at[isrc.at[0]], buf, sem).wait()

    pltpu.async_copy(hr.at[isrc.at[0]], rows_a, gsem_a)

    def hstep(i, carry):
        k0 = 2 * i
        k1 = k0 + 1
        _gwait(rows_a, gsem_a)
        pltpu.async_copy(hr.at[isrc.at[k1]], rows_b, gsem_b)
        pltpu.sync_copy(rows_a, acc64.at[idst.at[k0]], add=True)
        _gwait(rows_b, gsem_b)
        k2 = jnp.minimum(k0 + 2, NCH_H - 1)
        pltpu.async_copy(hr.at[isrc.at[k2]], rows_a, gsem_a)
        pltpu.sync_copy(rows_b, acc64.at[idst.at[k1]], add=True)
        return carry

    lax.fori_loop(0, NCH_H // 2, hstep, 0, unroll=False)
    _gwait(rows_a, gsem_a)

    def pstep(k, carry):
        pltpu.sync_copy(hpx.at[psrc.at[k]], prow)
        pltpu.sync_copy(prow, acc80.at[pdst.at[k]], add=True)
        return carry

    lax.fori_loop(0, NCH_P, pstep, 0, unroll=False)
    plsc.subcore_barrier()

    r0 = s * OROWS

    @pl.when(s < 15)
    def _():
        pltpu.sync_copy(acc64.at[pl.ds(r0, OROWS)], hio.at[c, pl.ds(r0, OROWS)])
        pltpu.sync_copy(acc80.at[pl.ds(r0, OROWS)], sp.at[c, pl.ds(r0, OROWS)])

    @pl.when(s == 15)
    def _():
        pltpu.sync_copy(acc64.at[pl.ds(15 * OROWS, 640)],
                        hio.at[c, pl.ds(15 * OROWS, 640)])
        pltpu.sync_copy(acc80.at[pl.ds(15 * OROWS, 640)],
                        sp.at[c, pl.ds(15 * OROWS, 640)])


@functools.partial(
    pl.kernel,
    out_type=jax.ShapeDtypeStruct((2, NRT, HH), jnp.float32),
    mesh=_mesh,
    compiler_params=_sc_params,
    scratch_types=[
        pltpu.VMEM_SHARED((ACC_ROWS, HH), jnp.float32),
        pltpu.VMEM((NCH_H, CHUNK), jnp.int32),
        pltpu.VMEM((NCH_H, CHUNK), jnp.int32),
        pltpu.VMEM((CHUNK, HH), jnp.float32),
        pltpu.VMEM((CHUNK, HH), jnp.float32),
        pltpu.SemaphoreType.DMA,
        pltpu.SemaphoreType.DMA,
    ],
)
def _sc_bin_routers(mio, eo_r, ei_r, z32, mr, acc32, isrc, idst,
                    rows_a, rows_b, gsem_a, gsem_b):
    c = lax.axis_index("c")
    s = lax.axis_index("s")
    z0 = s * OROWS

    @pl.when(s < 15)
    def _():
        pltpu.sync_copy(z32.at[pl.ds(z0, OROWS)], acc32.at[pl.ds(z0, OROWS)])

    @pl.when(s == 15)
    def _():
        pltpu.sync_copy(z32.at[pl.ds(15 * OROWS, 640)],
                        acc32.at[pl.ds(15 * OROWS, 640)])

    @pl.when(c == 0)
    def _():
        pltpu.sync_copy(ei_r.at[0, s], isrc)
        pltpu.sync_copy(ei_r.at[1, s], idst)

    @pl.when(c == 1)
    def _():
        pltpu.sync_copy(eo_r.at[1, s], isrc)
        pltpu.sync_copy(eo_r.at[0, s], idst)

    plsc.subcore_barrier()
    tb = mio.at[c]

    def _gwait(buf, sem):
        pltpu.make_async_copy(tb.at[isrc.at[0]], buf, sem).wait()

    pltpu.async_copy(tb.at[isrc.at[0]], rows_a, gsem_a)

    def step(i, carry):
        k0 = 2 * i
        k1 = k0 + 1
        _gwait(rows_a, gsem_a)
        pltpu.async_copy(tb.at[isrc.at[k1]], rows_b, gsem_b)
        pltpu.sync_copy(rows_a, acc32.at[idst.at[k0]], add=True)
        _gwait(rows_b, gsem_b)
        k2 = jnp.minimum(k0 + 2, NCH_H - 1)
        pltpu.async_copy(tb.at[isrc.at[k2]], rows_a, gsem_a)
        pltpu.sync_copy(rows_b, acc32.at[idst.at[k1]], add=True)
        return carry

    lax.fori_loop(0, NCH_H // 2, step, 0, unroll=False)
    _gwait(rows_a, gsem_a)
    plsc.subcore_barrier()

    r0 = s * OROWS

    @pl.when(s < 15)
    def _():
        pltpu.sync_copy(acc32.at[pl.ds(r0, OROWS)], mr.at[c, pl.ds(r0, OROWS)])

    @pl.when(s == 15)
    def _():
        pltpu.sync_copy(acc32.at[pl.ds(15 * OROWS, 640)],
                        mr.at[c, pl.ds(15 * OROWS, 640)])


TCT = 1000

_RX_NP = None
_FX_NP = None


def _expand_fold_mats():
    global _RX_NP, _FX_NP
    if _RX_NP is None:
        import numpy as _np
        r = _np.zeros((H, H * HH), _np.float32)
        f = _np.zeros((H * HH, HH), _np.float32)
        for i in range(H):
            r[i, i * HH:(i + 1) * HH] = 1.0
            f[i * HH:(i + 1) * HH, :] = _np.eye(HH, dtype=_np.float32)
        _RX_NP, _FX_NP = r, f
    return _RX_NP, _FX_NP


def _tree_fold(p):
    w = H * HH
    while w > HH:
        h = w // 2
        p = p[:, :h] + p[:, h:w]
        w = h
    return p


def _tc_dense_body(hio, sp, wp, bp3, rx, fx, mio):
    sext = sp[0] + sp[1]
    sv = sext[:, :H]
    cnt = sext[:, H:H + 1]
    e = jnp.dot(sv, wp[...], preferred_element_type=jnp.float32)
    hin_x = jnp.dot(hio[0], rx[...], preferred_element_type=jnp.float32)
    hout_x = jnp.dot(hio[1], rx[...], preferred_element_type=jnp.float32)
    mio[0] = _tree_fold(e * hin_x) + cnt * jnp.dot(
        hio[0], bp3[...], preferred_element_type=jnp.float32)
    mio[1] = _tree_fold(e * hout_x) + cnt * jnp.dot(
        hio[1], bp3[...], preferred_element_type=jnp.float32)


def _tc_dense(hio, sp, wp, bp2):
    rx, fx = _expand_fold_mats()
    return pl.pallas_call(
        _tc_dense_body,
        grid=(NCH // TCT,),
        in_specs=[
            pl.BlockSpec((2, TCT, H), lambda t: (0, t, 0)),
            pl.BlockSpec((2, TCT, H + 16), lambda t: (0, t, 0)),
            pl.BlockSpec((H, H * HH), lambda t: (0, 0)),
            pl.BlockSpec((H, HH), lambda t: (0, 0)),
            pl.BlockSpec((H, H * HH), lambda t: (0, 0)),
            pl.BlockSpec((H * HH, HH), lambda t: (0, 0)),
        ],
        out_specs=pl.BlockSpec((2, TCT, HH), lambda t: (0, t, 0)),
        out_shape=jax.ShapeDtypeStruct((2, NCH, HH), jnp.float32),
    )(hio, sp, wp, bp2, jnp.asarray(rx), jnp.asarray(fx))


TCR = 2000


def _tc_relu_body(mr, om):
    om[...] = jnp.maximum(
        jnp.concatenate([mr[0], mr[1]], axis=1), 0.0)


def _tc_relu(mr):
    return pl.pallas_call(
        _tc_relu_body,
        grid=(NRT // TCR,),
        in_specs=[pl.BlockSpec((2, TCR, HH), lambda t: (0, t, 0))],
        out_specs=pl.BlockSpec((TCR, H), lambda t: (t, 0)),
        out_shape=jax.ShapeDtypeStruct((NRT, H), jnp.float32),
    )(mr)


def kernel(h_router, h_packet, W_p, b_p, W_c, b_c, edge_output, edge_input, edge_pass):
    del W_c, b_c

    hpx = jnp.concatenate(
        [h_packet,
         jnp.ones((NPK, 1), jnp.float32),
         jnp.zeros((NPK, 15), jnp.float32)], axis=1)

    eo_r = edge_output.reshape(2, 16, NCH_H, CHUNK)
    ei_r = edge_input.reshape(2, 16, NCH_H, CHUNK)
    ep_r = edge_pass.reshape(2, 2, 16, NCH_P, CHUNK_P)

    z64 = jnp.zeros((ACC_ROWS, H), jnp.float32)
    z80 = jnp.zeros((ACC_ROWS, H + 16), jnp.float32)
    z32 = jnp.zeros((ACC_ROWS, HH), jnp.float32)

    hio, sp = _sc_bin_channels(h_router, hpx, eo_r, ei_r, ep_r, z64, z80)

    mio = _tc_dense(hio, sp, W_p, b_p.reshape(H, HH))

    mr = _sc_bin_routers(mio, eo_r, ei_r, z32)
    return _tc_relu(mr)


# speedup vs baseline: 31.0398x; 1.0519x over previous
import functools
import jax
import jax.numpy as jnp
from jax import lax
from jax.experimental import pallas as pl
from jax.experimental.pallas import tpu as pltpu
from jax.experimental.pallas import tpu_sc as plsc

H = 64
HH = 32
NCH = 10000
NRT = 10000
NPK = 10000
ACC_ROWS = 10000
OROWS = 624

CHUNK = 125
NCH_H = 50
NCH_P = 5

_mesh = plsc.VectorSubcoreMesh(core_axis_name="c", subcore_axis_name="s")
_sc_params = pltpu.CompilerParams(use_tc_tiling_on_sc=False)


def _zero_vmem(ref, nrows, width):
    z16 = jnp.zeros((16,), jnp.float32)

    def body(r, carry):
        for l in range(width // 16):
            ref[r, pl.ds(l * 16, 16)] = z16
        return carry

    lax.fori_loop(0, nrows, body, 0, unroll=False)


def _fill_acc(src, acc, s):
    for k in range(5):
        pltpu.sync_copy(src, acc.at[pl.ds(s * 625 + k * CHUNK, CHUNK)])


@functools.partial(
    pl.kernel,
    out_type=[
        jax.ShapeDtypeStruct((2, NCH, H), jnp.float32),
        jax.ShapeDtypeStruct((2, NCH, H), jnp.float32),
        jax.ShapeDtypeStruct((2, NCH, 16), jnp.float32),
    ],
    mesh=_mesh,
    compiler_params=_sc_params,
    scratch_types=[
        pltpu.VMEM_SHARED((ACC_ROWS, H), jnp.float32),
        pltpu.VMEM_SHARED((ACC_ROWS, H), jnp.float32),
        pltpu.VMEM_SHARED((ACC_ROWS, 16), jnp.float32),
        pltpu.VMEM((NCH_H, CHUNK), jnp.int32),
        pltpu.VMEM((NCH_H, CHUNK), jnp.int32),
        pltpu.VMEM((NCH_P, CHUNK), jnp.int32),
        pltpu.VMEM((NCH_P, CHUNK), jnp.int32),
        pltpu.VMEM((CHUNK, H), jnp.float32),
        pltpu.VMEM((CHUNK, H), jnp.float32),
        pltpu.VMEM((CHUNK, 16), jnp.float32),
        pltpu.VMEM((CHUNK, 16), jnp.float32),
        pltpu.SemaphoreType.DMA,
        pltpu.SemaphoreType.DMA,
        pltpu.SemaphoreType.DMA,
    ],
)
def _sc_bin_channels(hr, hp, eo_r, ei_r, ep_r,
                     hio, sp, cp,
                     acc_h, acc_s, acc_c, isrc, idst, psrc, pdst,
                     rows_a, rows_b, z16b, ones16, gsem_a, gsem_b, csem):
    c = lax.axis_index("c")
    s = lax.axis_index("s")

    _zero_vmem(rows_a, CHUNK, H)
    _zero_vmem(z16b, CHUNK, 16)
    _fill_acc(rows_a, acc_h, s)
    _fill_acc(rows_a, acc_s, s)
    _fill_acc(z16b, acc_c, s)

    def fill_ones(r, carry):
        ones16[r, pl.ds(0, 16)] = jnp.ones((16,), jnp.float32)
        return carry

    lax.fori_loop(0, CHUNK, fill_ones, 0, unroll=False)

    @pl.when(c == 0)
    def _():
        pltpu.sync_copy(eo_r.at[0, s], isrc)
        pltpu.sync_copy(eo_r.at[1, s], idst)

    @pl.when(c == 1)
    def _():
        pltpu.sync_copy(ei_r.at[1, s], isrc)
        pltpu.sync_copy(ei_r.at[0, s], idst)

    pltpu.sync_copy(ep_r.at[0, c, s], psrc)
    pltpu.sync_copy(ep_r.at[1, c, s], pdst)
    plsc.subcore_barrier()

    for k in range(NCH_P):
        pltpu.async_copy(ones16, acc_c.at[pdst.at[k]], csem, add=True)

    def _gwait(tbl, buf, sem):
        pltpu.make_async_copy(tbl.at[isrc.at[0]], buf, sem).wait()

    pltpu.async_copy(hr.at[isrc.at[0]], rows_a, gsem_a)

    def hstep(i, carry):
        k0 = 2 * i
        k1 = k0 + 1
        _gwait(hr, rows_a, gsem_a)
        pltpu.async_copy(hr.at[isrc.at[k1]], rows_b, gsem_b)
        pltpu.sync_copy(rows_a, acc_h.at[idst.at[k0]], add=True)
        _gwait(hr, rows_b, gsem_b)
        k2 = jnp.minimum(k0 + 2, NCH_H - 1)
        pltpu.async_copy(hr.at[isrc.at[k2]], rows_a, gsem_a)
        pltpu.sync_copy(rows_b, acc_h.at[idst.at[k1]], add=True)
        return carry

    lax.fori_loop(0, NCH_H // 2, hstep, 0, unroll=False)
    _gwait(hr, rows_a, gsem_a)

    pltpu.async_copy(hp.at[psrc.at[0]], rows_a, gsem_a)

    def pstep(i, carry):
        k0 = 2 * i
        k1 = k0 + 1
        _gwait(hp, rows_a, gsem_a)
        pltpu.async_copy(hp.at[psrc.at[k1]], rows_b, gsem_b)
        pltpu.sync_copy(rows_a, acc_s.at[pdst.at[k0]], add=True)
        _gwait(hp, rows_b, gsem_b)
        k2 = jnp.minimum(k0 + 2, NCH_P - 1)
        pltpu.async_copy(hp.at[psrc.at[k2]], rows_a, gsem_a)
        pltpu.sync_copy(rows_b, acc_s.at[pdst.at[k1]], add=True)
        return carry

    lax.fori_loop(0, NCH_P // 2, pstep, 0, unroll=False)
    _gwait(hp, rows_a, gsem_a)
    pltpu.sync_copy(rows_a, acc_s.at[pdst.at[NCH_P - 1]], add=True)

    for k in range(NCH_P):
        pltpu.make_async_copy(ones16, acc_c.at[pdst.at[0]], csem).wait()

    plsc.subcore_barrier()

    r0 = s * OROWS

    @pl.when(s < 15)
    def _():
        pltpu.sync_copy(acc_h.at[pl.ds(r0, OROWS)], hio.at[c, pl.ds(r0, OROWS)])
        pltpu.sync_copy(acc_s.at[pl.ds(r0, OROWS)], sp.at[c, pl.ds(r0, OROWS)])
        pltpu.sync_copy(acc_c.at[pl.ds(r0, OROWS)], cp.at[c, pl.ds(r0, OROWS)])

    @pl.when(s == 15)
    def _():
        pltpu.sync_copy(acc_h.at[pl.ds(15 * OROWS, 640)],
                        hio.at[c, pl.ds(15 * OROWS, 640)])
        pltpu.sync_copy(acc_s.at[pl.ds(15 * OROWS, 640)],
                        sp.at[c, pl.ds(15 * OROWS, 640)])
        pltpu.sync_copy(acc_c.at[pl.ds(15 * OROWS, 640)],
                        cp.at[c, pl.ds(15 * OROWS, 640)])


@functools.partial(
    pl.kernel,
    out_type=jax.ShapeDtypeStruct((2, NRT, HH), jnp.float32),
    mesh=_mesh,
    compiler_params=_sc_params,
    scratch_types=[
        pltpu.VMEM_SHARED((ACC_ROWS, HH), jnp.float32),
        pltpu.VMEM((NCH_H, CHUNK), jnp.int32),
        pltpu.VMEM((NCH_H, CHUNK), jnp.int32),
        pltpu.VMEM((CHUNK, HH), jnp.float32),
        pltpu.VMEM((CHUNK, HH), jnp.float32),
        pltpu.SemaphoreType.DMA,
        pltpu.SemaphoreType.DMA,
    ],
)
def _sc_bin_routers(mio, eo_r, ei_r, mr, acc32, isrc, idst,
                    rows_a, rows_b, gsem_a, gsem_b):
    c = lax.axis_index("c")
    s = lax.axis_index("s")

    _zero_vmem(rows_a, CHUNK, HH)
    _fill_acc(rows_a, acc32, s)

    @pl.when(c == 0)
    def _():
        pltpu.sync_copy(ei_r.at[0, s], isrc)
        pltpu.sync_copy(ei_r.at[1, s], idst)

    @pl.when(c == 1)
    def _():
        pltpu.sync_copy(eo_r.at[1, s], isrc)
        pltpu.sync_copy(eo_r.at[0, s], idst)

    plsc.subcore_barrier()
    tb = mio.at[c]

    def _gwait(buf, sem):
        pltpu.make_async_copy(tb.at[isrc.at[0]], buf, sem).wait()

    pltpu.async_copy(tb.at[isrc.at[0]], rows_a, gsem_a)

    def step(i, carry):
        k0 = 2 * i
        k1 = k0 + 1
        _gwait(rows_a, gsem_a)
        pltpu.async_copy(tb.at[isrc.at[k1]], rows_b, gsem_b)
        pltpu.sync_copy(rows_a, acc32.at[idst.at[k0]], add=True)
        _gwait(rows_b, gsem_b)
        k2 = jnp.minimum(k0 + 2, NCH_H - 1)
        pltpu.async_copy(tb.at[isrc.at[k2]], rows_a, gsem_a)
        pltpu.sync_copy(rows_b, acc32.at[idst.at[k1]], add=True)
        return carry

    lax.fori_loop(0, NCH_H // 2, step, 0, unroll=False)
    _gwait(rows_a, gsem_a)
    plsc.subcore_barrier()

    r0 = s * OROWS

    @pl.when(s < 15)
    def _():
        pltpu.sync_copy(acc32.at[pl.ds(r0, OROWS)], mr.at[c, pl.ds(r0, OROWS)])

    @pl.when(s == 15)
    def _():
        pltpu.sync_copy(acc32.at[pl.ds(15 * OROWS, 640)],
                        mr.at[c, pl.ds(15 * OROWS, 640)])


TCT = 1000

_RX_NP = None


def _expand_mat():
    global _RX_NP
    if _RX_NP is None:
        import numpy as _np
        r = _np.zeros((H, H * HH), _np.float32)
        for i in range(H):
            r[i, i * HH:(i + 1) * HH] = 1.0
        _RX_NP = r
    return _RX_NP


def _tree_fold(p):
    w = H * HH
    while w > HH:
        h = w // 2
        p = p[:, :h] + p[:, h:w]
        w = h
    return p


def _tc_dense_body(hio, sp, cp, wp, bp3, rx, mio):
    sv = sp[0] + sp[1]
    cnt = cp[0, :, :1] + cp[1, :, :1]
    e = jnp.dot(sv, wp[...], preferred_element_type=jnp.float32)
    hin_x = jnp.dot(hio[0], rx[...], preferred_element_type=jnp.float32)
    hout_x = jnp.dot(hio[1], rx[...], preferred_element_type=jnp.float32)
    mio[0] = _tree_fold(e * hin_x) + cnt * jnp.dot(
        hio[0], bp3[...], preferred_element_type=jnp.float32)
    mio[1] = _tree_fold(e * hout_x) + cnt * jnp.dot(
        hio[1], bp3[...], preferred_element_type=jnp.float32)


def _tc_dense(hio, sp, cp, wp, bp3):
    rx = _expand_mat()
    return pl.pallas_call(
        _tc_dense_body,
        grid=(NCH // TCT,),
        in_specs=[
            pl.BlockSpec((2, TCT, H), lambda t: (0, t, 0)),
            pl.BlockSpec((2, TCT, H), lambda t: (0, t, 0)),
            pl.BlockSpec((2, TCT, 16), lambda t: (0, t, 0)),
            pl.BlockSpec((H, H * HH), lambda t: (0, 0)),
            pl.BlockSpec((H, HH), lambda t: (0, 0)),
            pl.BlockSpec((H, H * HH), lambda t: (0, 0)),
        ],
        out_specs=pl.BlockSpec((2, TCT, HH), lambda t: (0, t, 0)),
        out_shape=jax.ShapeDtypeStruct((2, NCH, HH), jnp.float32),
    )(hio, sp, cp, wp, bp3, jnp.asarray(rx))


TCR = 2000


def _tc_relu_body(mr, om):
    om[...] = jnp.maximum(
        jnp.concatenate([mr[0], mr[1]], axis=1), 0.0)


def _tc_relu(mr):
    return pl.pallas_call(
        _tc_relu_body,
        grid=(NRT // TCR,),
        in_specs=[pl.BlockSpec((2, TCR, HH), lambda t: (0, t, 0))],
        out_specs=pl.BlockSpec((TCR, H), lambda t: (t, 0)),
        out_shape=jax.ShapeDtypeStruct((NRT, H), jnp.float32),
    )(mr)


def kernel(h_router, h_packet, W_p, b_p, W_c, b_c, edge_output, edge_input, edge_pass):
    del W_c, b_c

    eo_r = edge_output.reshape(2, 16, NCH_H, CHUNK)
    ei_r = edge_input.reshape(2, 16, NCH_H, CHUNK)
    ep_r = edge_pass.reshape(2, 2, 16, NCH_P, CHUNK)

    hio, sp, cp = _sc_bin_channels(h_router, h_packet, eo_r, ei_r, ep_r)

    mio = _tc_dense(hio, sp, cp, W_p, b_p.reshape(H, HH))

    mr = _sc_bin_routers(mio, eo_r, ei_r)
    return _tc_relu(mr)
